# baseline (speedup 1.0000x reference)
import functools

import jax
import jax.numpy as jnp
from jax import lax
from jax.experimental import pallas as pl
from jax.experimental.pallas import tpu as pltpu
from jax.experimental.pallas import tpu_sc as plsc

N_NODES = 10000
N_EDGES = 320000
N_FEATS = 9
VOCAB = 100
HID = 128
NG = 64

NC, NS = 2, 16
NW = NC * NS
CH = 128
NPAD = 10112
ROWS_PER_TILE = NPAD // NS
K_EDGE = 80
K_PAIR = 24
DUMMY_NODE = N_NODES
TBL_ROWS = 904
DUMMY_TBL = 900
DEG_W = 16


@functools.cache
def _mesh():
    return plsc.VectorSubcoreMesh(
        core_axis_name="c", subcore_axis_name="s", num_cores=NC, num_subcores=NS)


def _zero_vmem(ref, nrows, ncols, val=0.0):
    v = jnp.full((16,), val, jnp.float32)

    @pl.loop(0, nrows)
    def _(r):
        for c in range(ncols // 16):
            ref[r, pl.ds(c * 16, 16)] = v


def _fill_rows(zbuf, dst, base, nrows):
    zr = zbuf.shape[0]
    full, rem = divmod(nrows, zr)
    for k in range(full):
        pltpu.sync_copy(zbuf, dst.at[pl.ds(base + k * zr, zr)])
    if rem:
        pltpu.sync_copy(zbuf.at[pl.ds(0, rem)], dst.at[pl.ds(base + full * zr, rem)])


def _stream_gather_scatter(data_hbm, acc, idx_slab, nchunks,
                           ring, gb0, gb1, semi0, semi1, semg0, semg1):
    pltpu.async_copy(idx_slab.at[0], ring.at[0], semi0)
    pltpu.async_copy(idx_slab.at[1], ring.at[1], semi1)
    pltpu.make_async_copy(idx_slab.at[0], ring.at[0], semi0).wait()
    pltpu.async_copy(data_hbm.at[ring.at[0, 0]], gb0, semg0)

    @pl.loop(0, nchunks, step=2)
    def _(j):
        for b in (0, 1):
            jj = j + b
            gb, semg = (gb0, semg0) if b == 0 else (gb1, semg1)
            ogb, osemg = (gb1, semg1) if b == 0 else (gb0, semg0)
            semi_s = semi0 if b == 0 else semi1
            semi_o = semi1 if b == 0 else semi0

            @pl.when(jj + 1 < nchunks)
            def _():
                pltpu.make_async_copy(
                    idx_slab.at[jj + 1], ring.at[1 - b], semi_o).wait()
                pltpu.async_copy(data_hbm.at[ring.at[1 - b, 0]], ogb, osemg)

            pltpu.make_async_copy(data_hbm.at[ring.at[b, 0]], gb, semg).wait()
            pltpu.sync_copy(gb, acc.at[ring.at[b, 1]], add=True)

            @pl.when(jj + 2 < nchunks)
            def _():
                pltpu.async_copy(idx_slab.at[jj + 2], ring.at[b], semi_s)


def _stream_gather_scatter_1buf(data_hbm, acc, idx_slab, nchunks,
                                ring, gb, semi0, semi1, semg):
    pltpu.async_copy(idx_slab.at[0], ring.at[0], semi0)
    pltpu.async_copy(idx_slab.at[1], ring.at[1], semi1)

    @pl.loop(0, nchunks, step=2)
    def _(j):
        for b in (0, 1):
            jj = j + b
            semi_s = semi0 if b == 0 else semi1
            pltpu.make_async_copy(idx_slab.at[jj], ring.at[b], semi_s).wait()
            pltpu.async_copy(data_hbm.at[ring.at[b, 0]], gb, semg).wait()
            pltpu.sync_copy(gb, acc.at[ring.at[b, 1]], add=True)

            @pl.when(jj + 2 < nchunks)
            def _():
                pltpu.async_copy(idx_slab.at[jj + 2], ring.at[b], semi_s)


def _embed_body(tbl_hbm, pair_hbm, e_out, ring, gb0, acc, semi0, semi1, semg0):
    ci = lax.axis_index("c")
    si = lax.axis_index("s")
    wid = ci * NS + si
    base = si * ROWS_PER_TILE
    _zero_vmem(gb0, CH, HID)
    _fill_rows(gb0, acc, base, ROWS_PER_TILE)
    plsc.subcore_barrier()
    _stream_gather_scatter_1buf(tbl_hbm, acc, pair_hbm.at[wid], K_PAIR,
                                ring, gb0, semi0, semi1, semg0)
    plsc.subcore_barrier()
    pltpu.sync_copy(acc.at[pl.ds(base, ROWS_PER_TILE)],
                    e_out.at[ci, pl.ds(base, ROWS_PER_TILE)])


@functools.cache
def _embed_kernel():
    return pl.kernel(
        _embed_body,
        out_type=jax.ShapeDtypeStruct((NC, NPAD, HID), jnp.float32),
        mesh=_mesh(),
        scratch_types=[
            pltpu.VMEM((2, 2, CH), jnp.int32),
            pltpu.VMEM((CH, HID), jnp.float32),
            pltpu.VMEM_SHARED((NPAD, HID), jnp.float32),
            pltpu.SemaphoreType.DMA,
            pltpu.SemaphoreType.DMA,
            pltpu.SemaphoreType.DMA,
        ],
    )


def _agg_body(g_hbm, edge_hbm, p_out,
              ring, gb0, gb1, acc, semi0, semi1, semg0, semg1):
    ci = lax.axis_index("c")
    si = lax.axis_index("s")
    wid = ci * NS + si
    base = si * ROWS_PER_TILE
    _zero_vmem(gb0, CH, HID)
    _fill_rows(gb0, acc, base, ROWS_PER_TILE)
    plsc.subcore_barrier()
    _stream_gather_scatter(g_hbm, acc, edge_hbm.at[wid], K_EDGE,
                           ring, gb0, gb1, semi0, semi1, semg0, semg1)
    plsc.subcore_barrier()
    pltpu.sync_copy(acc.at[pl.ds(base, ROWS_PER_TILE)],
                    p_out.at[ci, pl.ds(base, ROWS_PER_TILE)])


@functools.cache
def _agg_kernel():
    return pl.kernel(
        _agg_body,
        out_type=jax.ShapeDtypeStruct((NC, NPAD, HID), jnp.float32),
        mesh=_mesh(),
        scratch_types=[
            pltpu.VMEM((2, 2, CH), jnp.int32),
            pltpu.VMEM((CH, HID), jnp.float32),
            pltpu.VMEM((CH, HID), jnp.float32),
            pltpu.VMEM_SHARED((NPAD, HID), jnp.float32),
            pltpu.SemaphoreType.DMA,
            pltpu.SemaphoreType.DMA,
            pltpu.SemaphoreType.DMA,
            pltpu.SemaphoreType.DMA,
        ],
    )


_MM = dict(precision=lax.Precision.HIGHEST, preferred_element_type=jnp.float32)
_DN = (((1,), (0,)), ((), ()))
NBLK = 4
RBLK = NPAD // NBLK

_SPEC2 = pl.BlockSpec((2, RBLK, HID), lambda i: (0, i, 0))
_SPECD = pl.BlockSpec((2, RBLK, HID), lambda i: (0, i, 0))
_SPECR = pl.BlockSpec((RBLK, HID), lambda i: (i, 0))
_SPECW = pl.BlockSpec((HID, HID), lambda i: (0, 0))
_SPECB = pl.BlockSpec((1, HID), lambda i: (0, 0))


def _dinv_of(d_ref):
    deg = d_ref[0][:, 0:1] + d_ref[1][:, 0:1] + 1.0
    return lax.rsqrt(deg)


def _tc_first_body(e_ref, d_ref, w_ref, o_ref):
    h = e_ref[0] + e_ref[1]
    hw = lax.dot_general(h, w_ref[...], _DN, **_MM)
    o_ref[...] = _dinv_of(d_ref) * hw


def _tc_first(E, D, W0):
    return pl.pallas_call(
        _tc_first_body,
        grid=(NBLK,),
        in_specs=[_SPEC2, _SPECD, _SPECW],
        out_specs=_SPECR,
        out_shape=jax.ShapeDtypeStruct((NPAD, HID), jnp.float32),
    )(E, D, W0)


def _tc_step_body(p_ref, g_ref, d_ref, b_ref, w_ref, o_ref):
    dinv = _dinv_of(d_ref)
    h = jnp.maximum(dinv * (p_ref[0] + p_ref[1] + g_ref[...]) + b_ref[...], 0.0)
    o_ref[...] = dinv * lax.dot_general(h, w_ref[...], _DN, **_MM)


def _tc_step(P, g, D, b2d, Wn):
    return pl.pallas_call(
        _tc_step_body,
        grid=(NBLK,),
        in_specs=[_SPEC2, _SPECR, _SPECD, _SPECB, _SPECW],
        out_specs=_SPECR,
        out_shape=jax.ShapeDtypeStruct((NPAD, HID), jnp.float32),
    )(P, g, D, b2d, Wn)


def _tc_pool_body(p_ref, g_ref, d_ref, b_ref, bat_ref, s_ref, c_ref):
    i = pl.program_id(0)

    @pl.when(i == 0)
    def _():
        s_ref[...] = jnp.zeros_like(s_ref)
        c_ref[...] = jnp.zeros_like(c_ref)

    dinv = _dinv_of(d_ref)
    h = jnp.maximum(dinv * (p_ref[0] + p_ref[1] + g_ref[...]) + b_ref[...], 0.0)
    seg = lax.broadcasted_iota(jnp.int32, (RBLK, NG), 1)
    sel_t = (bat_ref[...] == seg).astype(jnp.float32)
    dn0 = (((0,), (0,)), ((), ()))
    s_ref[...] += lax.dot_general(sel_t, h, dn0, **_MM)
    c_ref[...] += lax.dot_general(sel_t, jnp.ones((RBLK, 1), jnp.float32),
                                  dn0, **_MM)


def _tc_pool(P, g, D, b2d, bat2d):
    return pl.pallas_call(
        _tc_pool_body,
        grid=(NBLK,),
        in_specs=[_SPEC2, _SPECR, _SPECD, _SPECB,
                  pl.BlockSpec((RBLK, 1), lambda i: (i, 0))],
        out_specs=[pl.BlockSpec((NG, HID), lambda i: (0, 0)),
                   pl.BlockSpec((NG, 1), lambda i: (0, 0))],
        out_shape=[jax.ShapeDtypeStruct((NG, HID), jnp.float32),
                   jax.ShapeDtypeStruct((NG, 1), jnp.float32)],
    )(P, g, D, b2d, bat2d)


def _tc_head_body(s_ref, c_ref, wo_ref, bo_ref, o_ref):
    rep = s_ref[...] / jnp.maximum(c_ref[...], 1.0)
    o_ref[...] = lax.dot_general(rep, wo_ref[...], _DN, **_MM) + bo_ref[...]


def _tc_head(sums, counts, Wo, bo2d):
    return pl.pallas_call(
        _tc_head_body,
        out_shape=jax.ShapeDtypeStruct((NG, HID), jnp.float32),
    )(sums, counts, Wo, bo2d)


def _pack_chunks(src_idx, dst_idx, total, nchunks, fill_src, fill_dst):
    pad = NW * nchunks * CH - total
    s = jnp.concatenate([src_idx, jnp.full((pad,), fill_src, jnp.int32)])
    d = jnp.concatenate([dst_idx, jnp.full((pad,), fill_dst, jnp.int32)])
    return jnp.stack([s.reshape(NW, nchunks, CH),
                      d.reshape(NW, nchunks, CH)], axis=2)


def kernel(x, edge_index, batch, embs, W, b, W_out, b_out):
    x = x.astype(jnp.int32)
    src = edge_index[0].astype(jnp.int32)
    dst = edge_index[1].astype(jnp.int32)

    fidx = (x + (jnp.arange(N_FEATS, dtype=jnp.int32) * VOCAB)[None, :]).reshape(-1)
    node = jnp.repeat(jnp.arange(N_NODES, dtype=jnp.int32), N_FEATS)
    pairs = _pack_chunks(fidx, node, N_NODES * N_FEATS, K_PAIR,
                         DUMMY_TBL, DUMMY_NODE)
    edges = _pack_chunks(src, dst, N_EDGES, K_EDGE, DUMMY_NODE, DUMMY_NODE)

    tbl = jnp.pad(embs.reshape(N_FEATS * VOCAB, HID),
                  ((0, TBL_ROWS - N_FEATS * VOCAB), (0, 0)))

    deg_edges = _pack_chunks(jnp.zeros_like(src), dst, N_EDGES, K_EDGE,
                             0, DUMMY_NODE)
    ones_tbl = jnp.ones((8, HID), jnp.float32)
    Dd = _agg_kernel()(ones_tbl, deg_edges)
    E = _embed_kernel()(tbl, pairs)
    g = _tc_first(E, Dd, W[0])
    batp = jnp.concatenate(
        [batch.astype(jnp.int32), jnp.full((NPAD - N_NODES,), NG, jnp.int32)]
    ).reshape(NPAD, 1)
    for l in range(3):
        P = _agg_kernel()(g, edges)
        if l < 2:
            g = _tc_step(P, g, Dd, b[l].reshape(1, HID), W[l + 1])
        else:
            sums, counts = _tc_pool(P, g, Dd, b[2].reshape(1, HID), batp)
            out = _tc_head(sums, counts, W_out, b_out.reshape(1, HID))
    return out


# speedup vs baseline: 5.6203x; 5.6203x over previous
import functools

import jax
import jax.numpy as jnp
from jax import lax
from jax.experimental import pallas as pl
from jax.experimental.pallas import tpu as pltpu
from jax.experimental.pallas import tpu_sc as plsc

N_NODES = 10000
N_EDGES = 320000
N_FEATS = 9
VOCAB = 100
HID = 128
NG = 64

NC, NS = 2, 16
NW = NC * NS
CH = 128
NPAD = 10112
ROWS_PER_TILE = NPAD // NS
K_EDGE = 80
K_PAIR = 24
DUMMY_NODE = N_NODES
TBL_ROWS = 904
DUMMY_TBL = 900
DEG_W = 16


@functools.cache
def _mesh():
    return plsc.VectorSubcoreMesh(
        core_axis_name="c", subcore_axis_name="s", num_cores=NC, num_subcores=NS)


def _zero_vmem(ref, nrows, ncols, val=0.0):
    v = jnp.full((16,), val, jnp.float32)

    @pl.loop(0, nrows)
    def _(r):
        for c in range(ncols // 16):
            ref[r, pl.ds(c * 16, 16)] = v


def _fill_rows(zbuf, dst, base, nrows):
    zr = zbuf.shape[0]
    full, rem = divmod(nrows, zr)
    for k in range(full):
        pltpu.sync_copy(zbuf, dst.at[pl.ds(base + k * zr, zr)])
    if rem:
        pltpu.sync_copy(zbuf.at[pl.ds(0, rem)], dst.at[pl.ds(base + full * zr, rem)])


def _stream_gather_scatter(data_hbm, acc, idx_slab, nchunks,
                           ring, gb0, gb1, semi0, semi1, semg0, semg1):
    pltpu.async_copy(idx_slab.at[0], ring.at[0], semi0)
    pltpu.async_copy(idx_slab.at[1], ring.at[1], semi1)
    pltpu.make_async_copy(idx_slab.at[0], ring.at[0], semi0).wait()
    pltpu.async_copy(data_hbm.at[ring.at[0, 0]], gb0, semg0)

    @pl.loop(0, nchunks, step=2)
    def _(j):
        for b in (0, 1):
            jj = j + b
            gb, semg = (gb0, semg0) if b == 0 else (gb1, semg1)
            ogb, osemg = (gb1, semg1) if b == 0 else (gb0, semg0)
            semi_s = semi0 if b == 0 else semi1
            semi_o = semi1 if b == 0 else semi0

            @pl.when(jj + 1 < nchunks)
            def _():
                pltpu.make_async_copy(
                    idx_slab.at[jj + 1], ring.at[1 - b], semi_o).wait()
                pltpu.async_copy(data_hbm.at[ring.at[1 - b, 0]], ogb, osemg)

            pltpu.make_async_copy(data_hbm.at[ring.at[b, 0]], gb, semg).wait()
            pltpu.sync_copy(gb, acc.at[ring.at[b, 1]], add=True)

            @pl.when(jj + 2 < nchunks)
            def _():
                pltpu.async_copy(idx_slab.at[jj + 2], ring.at[b], semi_s)


def _stream_gather_scatter_1buf(data_hbm, acc, idx_slab, nchunks,
                                ring, gb, semi0, semi1, semg):
    pltpu.async_copy(idx_slab.at[0], ring.at[0], semi0)
    pltpu.async_copy(idx_slab.at[1], ring.at[1], semi1)

    @pl.loop(0, nchunks, step=2)
    def _(j):
        for b in (0, 1):
            jj = j + b
            semi_s = semi0 if b == 0 else semi1
            pltpu.make_async_copy(idx_slab.at[jj], ring.at[b], semi_s).wait()
            pltpu.async_copy(data_hbm.at[ring.at[b, 0]], gb, semg).wait()
            pltpu.sync_copy(gb, acc.at[ring.at[b, 1]], add=True)

            @pl.when(jj + 2 < nchunks)
            def _():
                pltpu.async_copy(idx_slab.at[jj + 2], ring.at[b], semi_s)


def _embed_body(tbl_hbm, pair_hbm, e_out, ring, gb0, acc, semi0, semi1, semg0):
    ci = lax.axis_index("c")
    si = lax.axis_index("s")
    wid = ci * NS + si
    base = si * ROWS_PER_TILE
    _zero_vmem(gb0, CH, HID)
    _fill_rows(gb0, acc, base, ROWS_PER_TILE)
    plsc.subcore_barrier()
    _stream_gather_scatter_1buf(tbl_hbm, acc, pair_hbm.at[wid], K_PAIR,
                                ring, gb0, semi0, semi1, semg0)
    plsc.subcore_barrier()
    pltpu.sync_copy(acc.at[pl.ds(base, ROWS_PER_TILE)],
                    e_out.at[ci, pl.ds(base, ROWS_PER_TILE)])


@functools.cache
def _embed_kernel():
    return pl.kernel(
        _embed_body,
        out_type=jax.ShapeDtypeStruct((NC, NPAD, HID), jnp.float32),
        mesh=_mesh(),
        scratch_types=[
            pltpu.VMEM((2, 2, CH), jnp.int32),
            pltpu.VMEM((CH, HID), jnp.float32),
            pltpu.VMEM_SHARED((NPAD, HID), jnp.float32),
            pltpu.SemaphoreType.DMA,
            pltpu.SemaphoreType.DMA,
            pltpu.SemaphoreType.DMA,
        ],
    )


def _agg_body(g_hbm, edge_hbm, p_out,
              ring, gb0, gb1, acc, semi0, semi1, semg0, semg1):
    ci = lax.axis_index("c")
    si = lax.axis_index("s")
    wid = ci * NS + si
    base = si * ROWS_PER_TILE
    _zero_vmem(gb0, CH, HID)
    _fill_rows(gb0, acc, base, ROWS_PER_TILE)
    plsc.subcore_barrier()
    _stream_gather_scatter(g_hbm, acc, edge_hbm.at[wid], K_EDGE,
                           ring, gb0, gb1, semi0, semi1, semg0, semg1)
    plsc.subcore_barrier()
    pltpu.sync_copy(acc.at[pl.ds(base, ROWS_PER_TILE)],
                    p_out.at[ci, pl.ds(base, ROWS_PER_TILE)])


@functools.cache
def _agg_kernel():
    return pl.kernel(
        _agg_body,
        out_type=jax.ShapeDtypeStruct((NC, NPAD, HID), jnp.float32),
        mesh=_mesh(),
        scratch_types=[
            pltpu.VMEM((2, 2, CH), jnp.int32),
            pltpu.VMEM((CH, HID), jnp.float32),
            pltpu.VMEM((CH, HID), jnp.float32),
            pltpu.VMEM_SHARED((NPAD, HID), jnp.float32),
            pltpu.SemaphoreType.DMA,
            pltpu.SemaphoreType.DMA,
            pltpu.SemaphoreType.DMA,
            pltpu.SemaphoreType.DMA,
        ],
    )


_MM = dict(precision=lax.Precision.HIGHEST, preferred_element_type=jnp.float32)
_DN = (((1,), (0,)), ((), ()))
NBLK = 4
RBLK = NPAD // NBLK

_SPEC2 = pl.BlockSpec((2, RBLK, HID), lambda i: (0, i, 0))
_SPECD = pl.BlockSpec((2, RBLK, HID), lambda i: (0, i, 0))
_SPECR = pl.BlockSpec((RBLK, HID), lambda i: (i, 0))
_SPECW = pl.BlockSpec((HID, HID), lambda i: (0, 0))
_SPECB = pl.BlockSpec((1, HID), lambda i: (0, 0))


def _dinv_of(d_ref):
    deg = d_ref[0][:, 0:1] + d_ref[1][:, 0:1] + 1.0
    return lax.rsqrt(deg)


def _tc_first_body(e_ref, d_ref, w_ref, o_ref):
    h = e_ref[0] + e_ref[1]
    hw = lax.dot_general(h, w_ref[...], _DN, **_MM)
    o_ref[...] = _dinv_of(d_ref) * hw


def _tc_first(E, D, W0):
    return pl.pallas_call(
        _tc_first_body,
        grid=(NBLK,),
        in_specs=[_SPEC2, _SPECD, _SPECW],
        out_specs=_SPECR,
        out_shape=jax.ShapeDtypeStruct((NPAD, HID), jnp.float32),
    )(E, D, W0)


def _tc_step_body(p_ref, g_ref, d_ref, b_ref, w_ref, o_ref):
    dinv = _dinv_of(d_ref)
    h = jnp.maximum(dinv * (p_ref[0] + p_ref[1] + g_ref[...]) + b_ref[...], 0.0)
    o_ref[...] = dinv * lax.dot_general(h, w_ref[...], _DN, **_MM)


def _tc_step(P, g, D, b2d, Wn):
    return pl.pallas_call(
        _tc_step_body,
        grid=(NBLK,),
        in_specs=[_SPEC2, _SPECR, _SPECD, _SPECB, _SPECW],
        out_specs=_SPECR,
        out_shape=jax.ShapeDtypeStruct((NPAD, HID), jnp.float32),
    )(P, g, D, b2d, Wn)


def _tc_pool_body(p_ref, g_ref, d_ref, b_ref, bat_ref, s_ref, c_ref):
    i = pl.program_id(0)

    @pl.when(i == 0)
    def _():
        s_ref[...] = jnp.zeros_like(s_ref)
        c_ref[...] = jnp.zeros_like(c_ref)

    dinv = _dinv_of(d_ref)
    h = jnp.maximum(dinv * (p_ref[0] + p_ref[1] + g_ref[...]) + b_ref[...], 0.0)
    seg = lax.broadcasted_iota(jnp.int32, (RBLK, NG), 1)
    sel_t = (bat_ref[...] == seg).astype(jnp.float32)
    dn0 = (((0,), (0,)), ((), ()))
    s_ref[...] += lax.dot_general(sel_t, h, dn0, **_MM)
    c_ref[...] += lax.dot_general(sel_t, jnp.ones((RBLK, 1), jnp.float32),
                                  dn0, **_MM)


def _tc_pool(P, g, D, b2d, bat2d):
    return pl.pallas_call(
        _tc_pool_body,
        grid=(NBLK,),
        in_specs=[_SPEC2, _SPECR, _SPECD, _SPECB,
                  pl.BlockSpec((RBLK, 1), lambda i: (i, 0))],
        out_specs=[pl.BlockSpec((NG, HID), lambda i: (0, 0)),
                   pl.BlockSpec((NG, 1), lambda i: (0, 0))],
        out_shape=[jax.ShapeDtypeStruct((NG, HID), jnp.float32),
                   jax.ShapeDtypeStruct((NG, 1), jnp.float32)],
    )(P, g, D, b2d, bat2d)


def _tc_head_body(s_ref, c_ref, wo_ref, bo_ref, o_ref):
    rep = s_ref[...] / jnp.maximum(c_ref[...], 1.0)
    o_ref[...] = lax.dot_general(rep, wo_ref[...], _DN, **_MM) + bo_ref[...]


def _tc_head(sums, counts, Wo, bo2d):
    return pl.pallas_call(
        _tc_head_body,
        out_shape=jax.ShapeDtypeStruct((NG, HID), jnp.float32),
    )(sums, counts, Wo, bo2d)


def _pack_chunks(src_idx, dst_idx, total, nchunks, fill_src, fill_dst):
    pad = NW * nchunks * CH - total
    s = jnp.concatenate([src_idx, jnp.full((pad,), fill_src, jnp.int32)])
    d = jnp.concatenate([dst_idx, jnp.full((pad,), fill_dst, jnp.int32)])
    return jnp.stack([s.reshape(NW, nchunks, CH),
                      d.reshape(NW, nchunks, CH)], axis=2)


def kernel(x, edge_index, batch, embs, W, b, W_out, b_out):
    x = x.astype(jnp.int32)
    src = edge_index[0].astype(jnp.int32)
    dst = edge_index[1].astype(jnp.int32)

    fidx = (x + (jnp.arange(N_FEATS, dtype=jnp.int32) * VOCAB)[None, :]).reshape(-1)
    node = jnp.repeat(jnp.arange(N_NODES, dtype=jnp.int32), N_FEATS)
    pairs = _pack_chunks(fidx, node, N_NODES * N_FEATS, K_PAIR,
                         DUMMY_TBL, DUMMY_NODE)
    edges = _pack_chunks(src, dst, N_EDGES, K_EDGE, DUMMY_NODE, DUMMY_NODE)

    tbl = jnp.pad(embs.reshape(N_FEATS * VOCAB, HID),
                  ((0, TBL_ROWS - N_FEATS * VOCAB), (0, 0)))

    Dd = _agg_kernel()(jnp.ones((NPAD, HID), jnp.float32), edges)
    E = _embed_kernel()(tbl, pairs)
    g = _tc_first(E, Dd, W[0])
    batp = jnp.concatenate(
        [batch.astype(jnp.int32), jnp.full((NPAD - N_NODES,), NG, jnp.int32)]
    ).reshape(NPAD, 1)
    for l in range(3):
        P = _agg_kernel()(g, edges)
        if l < 2:
            g = _tc_step(P, g, Dd, b[l].reshape(1, HID), W[l + 1])
        else:
            sums, counts = _tc_pool(P, g, Dd, b[2].reshape(1, HID), batp)
            out = _tc_head(sums, counts, W_out, b_out.reshape(1, HID))
    return out


# speedup vs baseline: 22.2982x; 3.9674x over previous
import functools

import jax
import jax.numpy as jnp
from jax import lax
from jax.experimental import pallas as pl
from jax.experimental.pallas import tpu as pltpu
from jax.experimental.pallas import tpu_sc as plsc

N_NODES = 10000
N_EDGES = 320000
N_FEATS = 9
VOCAB = 100
HID = 128
NG = 64

NC, NS = 2, 16
NW = NC * NS
CH = 128
NPAD = 10112
ROWS_PER_TILE = NPAD // NS
K_EDGE = 80
K_PAIR = 24
DUMMY_NODE = N_NODES
TBL_ROWS = 1024
DUMMY_TBL = 900
DEG_W = 16


@functools.cache
def _mesh():
    return plsc.VectorSubcoreMesh(
        core_axis_name="c", subcore_axis_name="s", num_cores=NC, num_subcores=NS)


def _zero_vmem(ref, nrows, ncols, val=0.0):
    v = jnp.full((16,), val, jnp.float32)

    @pl.loop(0, nrows)
    def _(r):
        for c in range(ncols // 16):
            ref[r, pl.ds(c * 16, 16)] = v


def _fill_rows(zbuf, dst, base, nrows):
    zr = zbuf.shape[0]
    full, rem = divmod(nrows, zr)
    for k in range(full):
        pltpu.sync_copy(zbuf, dst.at[pl.ds(base + k * zr, zr)])
    if rem:
        pltpu.sync_copy(zbuf.at[pl.ds(0, rem)], dst.at[pl.ds(base + full * zr, rem)])


def _stream_gather_scatter(data_hbm, acc, idx_slab, nchunks,
                           ring, gb0, gb1, semi0, semi1, semg0, semg1):
    pltpu.async_copy(idx_slab.at[0], ring.at[0], semi0)
    pltpu.async_copy(idx_slab.at[1], ring.at[1], semi1)
    pltpu.make_async_copy(idx_slab.at[0], ring.at[0], semi0).wait()
    pltpu.async_copy(data_hbm.at[ring.at[0, 0]], gb0, semg0)

    @pl.loop(0, nchunks, step=2)
    def _(j):
        for b in (0, 1):
            jj = j + b
            gb, semg = (gb0, semg0) if b == 0 else (gb1, semg1)
            ogb, osemg = (gb1, semg1) if b == 0 else (gb0, semg0)
            semi_s = semi0 if b == 0 else semi1
            semi_o = semi1 if b == 0 else semi0

            @pl.when(jj + 1 < nchunks)
            def _():
                pltpu.make_async_copy(
                    idx_slab.at[jj + 1], ring.at[1 - b], semi_o).wait()
                pltpu.async_copy(data_hbm.at[ring.at[1 - b, 0]], ogb, osemg)

            pltpu.make_async_copy(data_hbm.at[ring.at[b, 0]], gb, semg).wait()
            pltpu.sync_copy(gb, acc.at[ring.at[b, 1]], add=True)

            @pl.when(jj + 2 < nchunks)
            def _():
                pltpu.async_copy(idx_slab.at[jj + 2], ring.at[b], semi_s)


def _stream_gather_scatter_1buf(data_hbm, acc, idx_slab, nchunks,
                                ring, gb, semi0, semi1, semg):
    pltpu.async_copy(idx_slab.at[0], ring.at[0], semi0)
    pltpu.async_copy(idx_slab.at[1], ring.at[1], semi1)

    @pl.loop(0, nchunks, step=2)
    def _(j):
        for b in (0, 1):
            jj = j + b
            semi_s = semi0 if b == 0 else semi1
            pltpu.make_async_copy(idx_slab.at[jj], ring.at[b], semi_s).wait()
            pltpu.async_copy(data_hbm.at[ring.at[b, 0]], gb, semg).wait()
            pltpu.sync_copy(gb, acc.at[ring.at[b, 1]], add=True)

            @pl.when(jj + 2 < nchunks)
            def _():
                pltpu.async_copy(idx_slab.at[jj + 2], ring.at[b], semi_s)


def _embed_body(tbl_hbm, pair_hbm, e_out, ring, gb0, acc, semi0, semi1, semg0):
    ci = lax.axis_index("c")
    si = lax.axis_index("s")
    wid = ci * NS + si
    base = si * ROWS_PER_TILE
    _zero_vmem(gb0, CH, HID)
    _fill_rows(gb0, acc, base, ROWS_PER_TILE)
    plsc.subcore_barrier()
    _stream_gather_scatter_1buf(tbl_hbm, acc, pair_hbm.at[wid], K_PAIR,
                                ring, gb0, semi0, semi1, semg0)
    plsc.subcore_barrier()
    pltpu.sync_copy(acc.at[pl.ds(base, ROWS_PER_TILE)],
                    e_out.at[ci, pl.ds(base, ROWS_PER_TILE)])


@functools.cache
def _embed_kernel():
    return pl.kernel(
        _embed_body,
        out_type=jax.ShapeDtypeStruct((NC, NPAD, HID), jnp.float32),
        mesh=_mesh(),
        scratch_types=[
            pltpu.VMEM((2, 2, CH), jnp.int32),
            pltpu.VMEM((CH, HID), jnp.float32),
            pltpu.VMEM_SHARED((NPAD, HID), jnp.float32),
            pltpu.SemaphoreType.DMA,
            pltpu.SemaphoreType.DMA,
            pltpu.SemaphoreType.DMA,
        ],
    )


def _agg_body(g_hbm, edge_hbm, p_out,
              ring, gb0, gb1, acc, semi0, semi1, semg0, semg1):
    ci = lax.axis_index("c")
    si = lax.axis_index("s")
    wid = ci * NS + si
    base = si * ROWS_PER_TILE
    _zero_vmem(gb0, CH, HID)
    _fill_rows(gb0, acc, base, ROWS_PER_TILE)
    plsc.subcore_barrier()
    _stream_gather_scatter(g_hbm, acc, edge_hbm.at[wid], K_EDGE,
                           ring, gb0, gb1, semi0, semi1, semg0, semg1)
    plsc.subcore_barrier()
    pltpu.sync_copy(acc.at[pl.ds(base, ROWS_PER_TILE)],
                    p_out.at[ci, pl.ds(base, ROWS_PER_TILE)])


@functools.cache
def _agg_kernel():
    return pl.kernel(
        _agg_body,
        out_type=jax.ShapeDtypeStruct((NC, NPAD, HID), jnp.float32),
        mesh=_mesh(),
        scratch_types=[
            pltpu.VMEM((2, 2, CH), jnp.int32),
            pltpu.VMEM((CH, HID), jnp.float32),
            pltpu.VMEM((CH, HID), jnp.float32),
            pltpu.VMEM_SHARED((NPAD, HID), jnp.float32),
            pltpu.SemaphoreType.DMA,
            pltpu.SemaphoreType.DMA,
            pltpu.SemaphoreType.DMA,
            pltpu.SemaphoreType.DMA,
        ],
    )


_MM = dict(precision=lax.Precision.HIGHEST, preferred_element_type=jnp.float32)
_DN = (((1,), (0,)), ((), ()))
NBLK = 4
RBLK = NPAD // NBLK

_SPEC2 = pl.BlockSpec((2, RBLK, HID), lambda i: (0, i, 0))
_SPECD = pl.BlockSpec((2, RBLK, HID), lambda i: (0, i, 0))
_SPECR = pl.BlockSpec((RBLK, HID), lambda i: (i, 0))
_SPECW = pl.BlockSpec((HID, HID), lambda i: (0, 0))
_SPECB = pl.BlockSpec((1, HID), lambda i: (0, 0))


def _dinv_of(d_ref):
    deg = d_ref[0][:, 0:1] + d_ref[1][:, 0:1] + 1.0
    return lax.rsqrt(deg)


def _tc_first_body(e_ref, d_ref, w_ref, o_ref):
    h = e_ref[0] + e_ref[1]
    hw = lax.dot_general(h, w_ref[...], _DN, **_MM)
    o_ref[...] = _dinv_of(d_ref) * hw


def _tc_first(E, D, W0):
    return pl.pallas_call(
        _tc_first_body,
        grid=(NBLK,),
        in_specs=[_SPEC2, _SPECD, _SPECW],
        out_specs=_SPECR,
        out_shape=jax.ShapeDtypeStruct((NPAD, HID), jnp.float32),
    )(E, D, W0)


def _tc_step_body(p_ref, g_ref, d_ref, b_ref, w_ref, o_ref):
    dinv = _dinv_of(d_ref)
    h = jnp.maximum(dinv * (p_ref[0] + p_ref[1] + g_ref[...]) + b_ref[...], 0.0)
    o_ref[...] = dinv * lax.dot_general(h, w_ref[...], _DN, **_MM)


def _tc_step(P, g, D, b2d, Wn):
    return pl.pallas_call(
        _tc_step_body,
        grid=(NBLK,),
        in_specs=[_SPEC2, _SPECR, _SPECD, _SPECB, _SPECW],
        out_specs=_SPECR,
        out_shape=jax.ShapeDtypeStruct((NPAD, HID), jnp.float32),
    )(P, g, D, b2d, Wn)


def _tc_pool_body(p_ref, g_ref, d_ref, b_ref, bat_ref, s_ref, c_ref):
    i = pl.program_id(0)

    @pl.when(i == 0)
    def _():
        s_ref[...] = jnp.zeros_like(s_ref)
        c_ref[...] = jnp.zeros_like(c_ref)

    dinv = _dinv_of(d_ref)
    h = jnp.maximum(dinv * (p_ref[0] + p_ref[1] + g_ref[...]) + b_ref[...], 0.0)
    seg = lax.broadcasted_iota(jnp.int32, (RBLK, NG), 1)
    sel_t = (bat_ref[...] == seg).astype(jnp.float32)
    dn0 = (((0,), (0,)), ((), ()))
    s_ref[...] += lax.dot_general(sel_t, h, dn0, **_MM)
    c_ref[...] += lax.dot_general(sel_t, jnp.ones((RBLK, 1), jnp.float32),
                                  dn0, **_MM)


def _tc_pool(P, g, D, b2d, bat2d):
    return pl.pallas_call(
        _tc_pool_body,
        grid=(NBLK,),
        in_specs=[_SPEC2, _SPECR, _SPECD, _SPECB,
                  pl.BlockSpec((RBLK, 1), lambda i: (i, 0))],
        out_specs=[pl.BlockSpec((NG, HID), lambda i: (0, 0)),
                   pl.BlockSpec((NG, 1), lambda i: (0, 0))],
        out_shape=[jax.ShapeDtypeStruct((NG, HID), jnp.float32),
                   jax.ShapeDtypeStruct((NG, 1), jnp.float32)],
    )(P, g, D, b2d, bat2d)


def _tc_head_body(s_ref, c_ref, wo_ref, bo_ref, o_ref):
    rep = s_ref[...] / jnp.maximum(c_ref[...], 1.0)
    o_ref[...] = lax.dot_general(rep, wo_ref[...], _DN, **_MM) + bo_ref[...]


def _tc_head(sums, counts, Wo, bo2d):
    return pl.pallas_call(
        _tc_head_body,
        out_shape=jax.ShapeDtypeStruct((NG, HID), jnp.float32),
    )(sums, counts, Wo, bo2d)


def _pack_chunks(src_idx, dst_idx, total, nchunks, fill_src_lo, fill_src_n):
    pad = NW * nchunks * CH - total
    cyc = jnp.arange(pad, dtype=jnp.int32)
    s = jnp.concatenate([src_idx, fill_src_lo + cyc % fill_src_n])
    d = jnp.concatenate([dst_idx, DUMMY_NODE + cyc % (NPAD - N_NODES)])
    return jnp.stack([s.reshape(NW, nchunks, CH),
                      d.reshape(NW, nchunks, CH)], axis=2)


def kernel(x, edge_index, batch, embs, W, b, W_out, b_out):
    x = x.astype(jnp.int32)
    src = edge_index[0].astype(jnp.int32)
    dst = edge_index[1].astype(jnp.int32)

    fidx = (x + (jnp.arange(N_FEATS, dtype=jnp.int32) * VOCAB)[None, :]).reshape(-1)
    node = jnp.repeat(jnp.arange(N_NODES, dtype=jnp.int32), N_FEATS)
    pairs = _pack_chunks(fidx, node, N_NODES * N_FEATS, K_PAIR,
                         DUMMY_TBL, TBL_ROWS - DUMMY_TBL)
    edges = _pack_chunks(src, dst, N_EDGES, K_EDGE,
                         DUMMY_NODE, NPAD - N_NODES)

    tbl = jnp.pad(embs.reshape(N_FEATS * VOCAB, HID),
                  ((0, TBL_ROWS - N_FEATS * VOCAB), (0, 0)))

    Dd = _agg_kernel()(jnp.ones((NPAD, HID), jnp.float32), edges)
    E = _embed_kernel()(tbl, pairs)
    g = _tc_first(E, Dd, W[0])
    batp = jnp.concatenate(
        [batch.astype(jnp.int32), jnp.full((NPAD - N_NODES,), NG, jnp.int32)]
    ).reshape(NPAD, 1)
    for l in range(3):
        P = _agg_kernel()(g, edges)
        if l < 2:
            g = _tc_step(P, g, Dd, b[l].reshape(1, HID), W[l + 1])
        else:
            sums, counts = _tc_pool(P, g, Dd, b[2].reshape(1, HID), batp)
            out = _tc_head(sums, counts, W_out, b_out.reshape(1, HID))
    return out


# speedup vs baseline: 22.9078x; 1.0273x over previous
import functools

import jax
import jax.numpy as jnp
from jax import lax
from jax.experimental import pallas as pl
from jax.experimental.pallas import tpu as pltpu
from jax.experimental.pallas import tpu_sc as plsc

N_NODES = 10000
N_EDGES = 320000
N_FEATS = 9
VOCAB = 100
HID = 128
NG = 64

NC, NS = 2, 16
NW = NC * NS
CH = 128
NPAD = 10112
ROWS_PER_TILE = NPAD // NS
K_EDGE = 80
K_PAIR = 24
DUMMY_NODE = N_NODES
TBL_ROWS = 1024
DUMMY_TBL = 900
DEG_W = 16


@functools.cache
def _mesh():
    return plsc.VectorSubcoreMesh(
        core_axis_name="c", subcore_axis_name="s", num_cores=NC, num_subcores=NS)


def _zero_vmem(ref, nrows, ncols, val=0.0):
    v = jnp.full((16,), val, jnp.float32)

    @pl.loop(0, nrows)
    def _(r):
        for c in range(ncols // 16):
            ref[r, pl.ds(c * 16, 16)] = v


def _fill_rows(zbuf, dst, base, nrows):
    zr = zbuf.shape[0]
    full, rem = divmod(nrows, zr)
    for k in range(full):
        pltpu.sync_copy(zbuf, dst.at[pl.ds(base + k * zr, zr)])
    if rem:
        pltpu.sync_copy(zbuf.at[pl.ds(0, rem)], dst.at[pl.ds(base + full * zr, rem)])


def _stream_gather_scatter(data_hbm, acc, idx_slab, nchunks,
                           ring, gb0, gb1, semi0, semi1, semg0, semg1):
    pltpu.async_copy(idx_slab.at[0], ring.at[0], semi0)
    pltpu.async_copy(idx_slab.at[1], ring.at[1], semi1)
    pltpu.make_async_copy(idx_slab.at[0], ring.at[0], semi0).wait()
    pltpu.async_copy(data_hbm.at[ring.at[0, 0]], gb0, semg0)

    @pl.loop(0, nchunks, step=2)
    def _(j):
        for b in (0, 1):
            jj = j + b
            gb, semg = (gb0, semg0) if b == 0 else (gb1, semg1)
            ogb, osemg = (gb1, semg1) if b == 0 else (gb0, semg0)
            semi_s = semi0 if b == 0 else semi1
            semi_o = semi1 if b == 0 else semi0

            @pl.when(jj + 1 < nchunks)
            def _():
                pltpu.make_async_copy(
                    idx_slab.at[jj + 1], ring.at[1 - b], semi_o).wait()
                pltpu.async_copy(data_hbm.at[ring.at[1 - b, 0]], ogb, osemg)

            pltpu.make_async_copy(data_hbm.at[ring.at[b, 0]], gb, semg).wait()
            pltpu.sync_copy(gb, acc.at[ring.at[b, 1]], add=True)

            @pl.when(jj + 2 < nchunks)
            def _():
                pltpu.async_copy(idx_slab.at[jj + 2], ring.at[b], semi_s)


def _embed_body(tbl_hbm, pair_hbm, e_out, ring, gb0, gb1, acc,
                semi0, semi1, semg0, semg1):
    ci = lax.axis_index("c")
    si = lax.axis_index("s")
    wid = ci * NS + si
    base = si * ROWS_PER_TILE
    _zero_vmem(gb0, CH, HID)
    _fill_rows(gb0, acc, base, ROWS_PER_TILE)
    plsc.subcore_barrier()
    _stream_gather_scatter(tbl_hbm, acc, pair_hbm.at[wid], K_PAIR,
                           ring, gb0, gb1, semi0, semi1, semg0, semg1)
    plsc.subcore_barrier()
    pltpu.sync_copy(acc.at[pl.ds(base, ROWS_PER_TILE)],
                    e_out.at[ci, pl.ds(base, ROWS_PER_TILE)])


@functools.cache
def _embed_kernel():
    return pl.kernel(
        _embed_body,
        out_type=jax.ShapeDtypeStruct((NC, NPAD, HID), jnp.float32),
        mesh=_mesh(),
        scratch_types=[
            pltpu.VMEM((2, 2, CH), jnp.int32),
            pltpu.VMEM((CH, HID), jnp.float32),
            pltpu.VMEM((CH, HID), jnp.float32),
            pltpu.VMEM_SHARED((NPAD, HID), jnp.float32),
            pltpu.SemaphoreType.DMA,
            pltpu.SemaphoreType.DMA,
            pltpu.SemaphoreType.DMA,
            pltpu.SemaphoreType.DMA,
        ],
    )


def _agg_body(g_hbm, edge_hbm, p_out,
              ring, gb0, gb1, acc, semi0, semi1, semg0, semg1):
    ci = lax.axis_index("c")
    si = lax.axis_index("s")
    wid = ci * NS + si
    base = si * ROWS_PER_TILE
    _zero_vmem(gb0, CH, HID)
    _fill_rows(gb0, acc, base, ROWS_PER_TILE)
    plsc.subcore_barrier()
    _stream_gather_scatter(g_hbm, acc, edge_hbm.at[wid], K_EDGE,
                           ring, gb0, gb1, semi0, semi1, semg0, semg1)
    plsc.subcore_barrier()
    pltpu.sync_copy(acc.at[pl.ds(base, ROWS_PER_TILE)],
                    p_out.at[ci, pl.ds(base, ROWS_PER_TILE)])


@functools.cache
def _agg_kernel():
    return pl.kernel(
        _agg_body,
        out_type=jax.ShapeDtypeStruct((NC, NPAD, HID), jnp.float32),
        mesh=_mesh(),
        scratch_types=[
            pltpu.VMEM((2, 2, CH), jnp.int32),
            pltpu.VMEM((CH, HID), jnp.float32),
            pltpu.VMEM((CH, HID), jnp.float32),
            pltpu.VMEM_SHARED((NPAD, HID), jnp.float32),
            pltpu.SemaphoreType.DMA,
            pltpu.SemaphoreType.DMA,
            pltpu.SemaphoreType.DMA,
            pltpu.SemaphoreType.DMA,
        ],
    )


_MM = dict(precision=lax.Precision.HIGHEST, preferred_element_type=jnp.float32)
_DN = (((1,), (0,)), ((), ()))
NBLK = 4
RBLK = NPAD // NBLK

_SPEC2 = pl.BlockSpec((2, RBLK, HID), lambda i: (0, i, 0))
_SPECD = pl.BlockSpec((2, RBLK, HID), lambda i: (0, i, 0))
_SPECR = pl.BlockSpec((RBLK, HID), lambda i: (i, 0))
_SPECW = pl.BlockSpec((HID, HID), lambda i: (0, 0))
_SPECB = pl.BlockSpec((1, HID), lambda i: (0, 0))


def _dinv_of(d_ref):
    deg = d_ref[0][:, 0:1] + d_ref[1][:, 0:1] + 1.0
    return lax.rsqrt(deg)


def _tc_first_body(e_ref, d_ref, w_ref, o_ref):
    h = e_ref[0] + e_ref[1]
    hw = lax.dot_general(h, w_ref[...], _DN, **_MM)
    o_ref[...] = _dinv_of(d_ref) * hw


def _tc_first(E, D, W0):
    return pl.pallas_call(
        _tc_first_body,
        grid=(NBLK,),
        in_specs=[_SPEC2, _SPECD, _SPECW],
        out_specs=_SPECR,
        out_shape=jax.ShapeDtypeStruct((NPAD, HID), jnp.float32),
    )(E, D, W0)


def _tc_step_body(p_ref, g_ref, d_ref, b_ref, w_ref, o_ref):
    dinv = _dinv_of(d_ref)
    h = jnp.maximum(dinv * (p_ref[0] + p_ref[1] + g_ref[...]) + b_ref[...], 0.0)
    o_ref[...] = dinv * lax.dot_general(h, w_ref[...], _DN, **_MM)


def _tc_step(P, g, D, b2d, Wn):
    return pl.pallas_call(
        _tc_step_body,
        grid=(NBLK,),
        in_specs=[_SPEC2, _SPECR, _SPECD, _SPECB, _SPECW],
        out_specs=_SPECR,
        out_shape=jax.ShapeDtypeStruct((NPAD, HID), jnp.float32),
    )(P, g, D, b2d, Wn)


def _tc_pool_body(p_ref, g_ref, d_ref, b_ref, bat_ref, wo_ref, bo_ref,
                  s_ref, c_ref, o_ref):
    i = pl.program_id(0)

    @pl.when(i == 0)
    def _():
        s_ref[...] = jnp.zeros_like(s_ref)
        c_ref[...] = jnp.zeros_like(c_ref)

    dinv = _dinv_of(d_ref)
    h = jnp.maximum(dinv * (p_ref[0] + p_ref[1] + g_ref[...]) + b_ref[...], 0.0)
    seg = lax.broadcasted_iota(jnp.int32, (RBLK, NG), 1)
    sel_t = (bat_ref[...] == seg).astype(jnp.float32)
    dn0 = (((0,), (0,)), ((), ()))
    s_ref[...] += lax.dot_general(sel_t, h, dn0, **_MM)
    c_ref[...] += lax.dot_general(sel_t, jnp.ones((RBLK, 1), jnp.float32),
                                  dn0, **_MM)

    @pl.when(i == NBLK - 1)
    def _():
        rep = s_ref[...] / jnp.maximum(c_ref[...], 1.0)
        o_ref[...] = lax.dot_general(rep, wo_ref[...], _DN, **_MM) + bo_ref[...]


def _tc_pool(P, g, D, b2d, bat2d, Wo, bo2d):
    _, _, out = pl.pallas_call(
        _tc_pool_body,
        grid=(NBLK,),
        in_specs=[_SPEC2, _SPECR, _SPECD, _SPECB,
                  pl.BlockSpec((RBLK, 1), lambda i: (i, 0)),
                  _SPECW, _SPECB],
        out_specs=[pl.BlockSpec((NG, HID), lambda i: (0, 0)),
                   pl.BlockSpec((NG, 1), lambda i: (0, 0)),
                   pl.BlockSpec((NG, HID), lambda i: (0, 0))],
        out_shape=[jax.ShapeDtypeStruct((NG, HID), jnp.float32),
                   jax.ShapeDtypeStruct((NG, 1), jnp.float32),
                   jax.ShapeDtypeStruct((NG, HID), jnp.float32)],
    )(P, g, D, b2d, bat2d, Wo, bo2d)
    return out


def _pack_chunks(src_idx, dst_idx, total, nchunks, fill_src_lo, fill_src_n):
    pad = NW * nchunks * CH - total
    cyc = jnp.arange(pad, dtype=jnp.int32)
    s = jnp.concatenate([src_idx, fill_src_lo + cyc % fill_src_n])
    d = jnp.concatenate([dst_idx, DUMMY_NODE + cyc % (NPAD - N_NODES)])
    return jnp.stack([s.reshape(NW, nchunks, CH),
                      d.reshape(NW, nchunks, CH)], axis=2)


def kernel(x, edge_index, batch, embs, W, b, W_out, b_out):
    x = x.astype(jnp.int32)
    src = edge_index[0].astype(jnp.int32)
    dst = edge_index[1].astype(jnp.int32)

    fidx = (x + (jnp.arange(N_FEATS, dtype=jnp.int32) * VOCAB)[None, :]).reshape(-1)
    node = jnp.repeat(jnp.arange(N_NODES, dtype=jnp.int32), N_FEATS)
    pairs = _pack_chunks(fidx, node, N_NODES * N_FEATS, K_PAIR,
                         DUMMY_TBL, TBL_ROWS - DUMMY_TBL)
    edges = _pack_chunks(src, dst, N_EDGES, K_EDGE,
                         DUMMY_NODE, NPAD - N_NODES)

    tbl = jnp.pad(embs.reshape(N_FEATS * VOCAB, HID),
                  ((0, TBL_ROWS - N_FEATS * VOCAB), (0, 0)))

    Dd = _agg_kernel()(jnp.ones((NPAD, HID), jnp.float32), edges)
    E = _embed_kernel()(tbl, pairs)
    g = _tc_first(E, Dd, W[0])
    batp = jnp.concatenate(
        [batch.astype(jnp.int32), jnp.full((NPAD - N_NODES,), NG, jnp.int32)]
    ).reshape(NPAD, 1)
    for l in range(3):
        P = _agg_kernel()(g, edges)
        if l < 2:
            g = _tc_step(P, g, Dd, b[l].reshape(1, HID), W[l + 1])
        else:
            out = _tc_pool(P, g, Dd, b[2].reshape(1, HID), batp,
                           W_out, b_out.reshape(1, HID))
    return out


# speedup vs baseline: 25.0936x; 1.0954x over previous
import functools

import jax
import jax.numpy as jnp
from jax import lax
from jax.experimental import pallas as pl
from jax.experimental.pallas import tpu as pltpu
from jax.experimental.pallas import tpu_sc as plsc

N_NODES = 10000
N_EDGES = 320000
N_FEATS = 9
VOCAB = 100
HID = 128
NG = 64

NC, NS = 2, 16
NW = NC * NS
CH = 128
NPAD = 10112
ROWS_PER_TILE = NPAD // NS
K_EDGE = 80
K_PAIR = 24
DUMMY_NODE = N_NODES
TBL_ROWS = 1024
DUMMY_TBL = 900
DEG_W = 16


@functools.cache
def _mesh():
    return plsc.VectorSubcoreMesh(
        core_axis_name="c", subcore_axis_name="s", num_cores=NC, num_subcores=NS)


def _zero_vmem(ref, nrows, ncols, val=0.0):
    v = jnp.full((16,), val, jnp.float32)

    @pl.loop(0, nrows)
    def _(r):
        for c in range(ncols // 16):
            ref[r, pl.ds(c * 16, 16)] = v


def _fill_rows(zbuf, dst, base, nrows):
    zr = zbuf.shape[0]
    full, rem = divmod(nrows, zr)
    for k in range(full):
        pltpu.sync_copy(zbuf, dst.at[pl.ds(base + k * zr, zr)])
    if rem:
        pltpu.sync_copy(zbuf.at[pl.ds(0, rem)], dst.at[pl.ds(base + full * zr, rem)])


def _stream_gather_scatter(data_hbm, acc, idx_slab, nchunks,
                           ring, gb0, gb1, semi0, semi1, semg0, semg1):
    pltpu.async_copy(idx_slab.at[0], ring.at[0], semi0)
    pltpu.async_copy(idx_slab.at[1], ring.at[1], semi1)
    pltpu.make_async_copy(idx_slab.at[0], ring.at[0], semi0).wait()
    pltpu.async_copy(data_hbm.at[ring.at[0, 0]], gb0, semg0)

    @pl.loop(0, nchunks, step=2)
    def _(j):
        for b in (0, 1):
            jj = j + b
            gb, semg = (gb0, semg0) if b == 0 else (gb1, semg1)
            ogb, osemg = (gb1, semg1) if b == 0 else (gb0, semg0)
            semi_s = semi0 if b == 0 else semi1
            semi_o = semi1 if b == 0 else semi0

            @pl.when(jj + 1 < nchunks)
            def _():
                pltpu.make_async_copy(
                    idx_slab.at[jj + 1], ring.at[1 - b], semi_o).wait()
                pltpu.async_copy(data_hbm.at[ring.at[1 - b, 0]], ogb, osemg)

            pltpu.make_async_copy(data_hbm.at[ring.at[b, 0]], gb, semg).wait()
            pltpu.sync_copy(gb, acc.at[ring.at[b, 1]], add=True)

            @pl.when(jj + 2 < nchunks)
            def _():
                pltpu.async_copy(idx_slab.at[jj + 2], ring.at[b], semi_s)


def _deg_body(edge_hbm, d_out, ring, onz, dacc, semj0, semj1):
    ci = lax.axis_index("c")
    si = lax.axis_index("s")
    wid = ci * NS + si
    base = si * ROWS_PER_TILE
    _zero_vmem(onz.at[0], CH, HID, val=1.0)
    _zero_vmem(onz.at[1], CH, HID)
    _fill_rows(onz.at[1], dacc, base, ROWS_PER_TILE)
    plsc.subcore_barrier()
    idx_slab = edge_hbm.at[wid]
    pltpu.async_copy(idx_slab.at[0], ring.at[0], semj0)
    pltpu.async_copy(idx_slab.at[1], ring.at[1], semj1)

    @pl.loop(0, K_EDGE, step=2)
    def _(j):
        for b in (0, 1):
            jj = j + b
            sem = semj0 if b == 0 else semj1
            pltpu.make_async_copy(idx_slab.at[jj], ring.at[b], sem).wait()
            pltpu.sync_copy(onz.at[0], dacc.at[ring.at[b, 1]], add=True)

            @pl.when(jj + 2 < K_EDGE)
            def _():
                pltpu.async_copy(idx_slab.at[jj + 2], ring.at[b], sem)

    plsc.subcore_barrier()
    pltpu.sync_copy(dacc.at[pl.ds(base, ROWS_PER_TILE)],
                    d_out.at[ci, pl.ds(base, ROWS_PER_TILE)])


@functools.cache
def _deg_kernel():
    return pl.kernel(
        _deg_body,
        out_type=jax.ShapeDtypeStruct((NC, NPAD, HID), jnp.float32),
        mesh=_mesh(),
        scratch_types=[
            pltpu.VMEM((2, 2, CH), jnp.int32),
            pltpu.VMEM((2, CH, HID), jnp.float32),
            pltpu.VMEM_SHARED((NPAD, HID), jnp.float32),
            pltpu.SemaphoreType.DMA,
            pltpu.SemaphoreType.DMA,
        ],
    )


def _embed_body(tbl_hbm, pair_hbm, e_out, ring, gb0, gb1, acc,
                semi0, semi1, semg0, semg1):
    ci = lax.axis_index("c")
    si = lax.axis_index("s")
    wid = ci * NS + si
    base = si * ROWS_PER_TILE
    _zero_vmem(gb0, CH, HID)
    _fill_rows(gb0, acc, base, ROWS_PER_TILE)
    plsc.subcore_barrier()
    _stream_gather_scatter(tbl_hbm, acc, pair_hbm.at[wid], K_PAIR,
                           ring, gb0, gb1, semi0, semi1, semg0, semg1)
    plsc.subcore_barrier()
    pltpu.sync_copy(acc.at[pl.ds(base, ROWS_PER_TILE)],
                    e_out.at[ci, pl.ds(base, ROWS_PER_TILE)])


@functools.cache
def _embed_kernel():
    return pl.kernel(
        _embed_body,
        out_type=jax.ShapeDtypeStruct((NC, NPAD, HID), jnp.float32),
        mesh=_mesh(),
        scratch_types=[
            pltpu.VMEM((2, 2, CH), jnp.int32),
            pltpu.VMEM((CH, HID), jnp.float32),
            pltpu.VMEM((CH, HID), jnp.float32),
            pltpu.VMEM_SHARED((NPAD, HID), jnp.float32),
            pltpu.SemaphoreType.DMA,
            pltpu.SemaphoreType.DMA,
            pltpu.SemaphoreType.DMA,
            pltpu.SemaphoreType.DMA,
        ],
    )


def _agg_body(g_hbm, edge_hbm, p_out,
              ring, gb0, gb1, acc, semi0, semi1, semg0, semg1):
    ci = lax.axis_index("c")
    si = lax.axis_index("s")
    wid = ci * NS + si
    base = si * ROWS_PER_TILE
    _zero_vmem(gb0, CH, HID)
    _fill_rows(gb0, acc, base, ROWS_PER_TILE)
    plsc.subcore_barrier()
    _stream_gather_scatter(g_hbm, acc, edge_hbm.at[wid], K_EDGE,
                           ring, gb0, gb1, semi0, semi1, semg0, semg1)
    plsc.subcore_barrier()
    pltpu.sync_copy(acc.at[pl.ds(base, ROWS_PER_TILE)],
                    p_out.at[ci, pl.ds(base, ROWS_PER_TILE)])


@functools.cache
def _agg_kernel():
    return pl.kernel(
        _agg_body,
        out_type=jax.ShapeDtypeStruct((NC, NPAD, HID), jnp.float32),
        mesh=_mesh(),
        scratch_types=[
            pltpu.VMEM((2, 2, CH), jnp.int32),
            pltpu.VMEM((CH, HID), jnp.float32),
            pltpu.VMEM((CH, HID), jnp.float32),
            pltpu.VMEM_SHARED((NPAD, HID), jnp.float32),
            pltpu.SemaphoreType.DMA,
            pltpu.SemaphoreType.DMA,
            pltpu.SemaphoreType.DMA,
            pltpu.SemaphoreType.DMA,
        ],
    )


_MM = dict(precision=lax.Precision.HIGHEST, preferred_element_type=jnp.float32)
_DN = (((1,), (0,)), ((), ()))
NBLK = 4
RBLK = NPAD // NBLK

_SPEC2 = pl.BlockSpec((2, RBLK, HID), lambda i: (0, i, 0))
_SPECD = pl.BlockSpec((2, RBLK, HID), lambda i: (0, i, 0))
_SPECR = pl.BlockSpec((RBLK, HID), lambda i: (i, 0))
_SPECW = pl.BlockSpec((HID, HID), lambda i: (0, 0))
_SPECB = pl.BlockSpec((1, HID), lambda i: (0, 0))


def _dinv_of(d_ref):
    deg = d_ref[0][:, 0:1] + d_ref[1][:, 0:1] + 1.0
    return lax.rsqrt(deg)


def _tc_first_body(e_ref, d_ref, w_ref, o_ref):
    h = e_ref[0] + e_ref[1]
    hw = lax.dot_general(h, w_ref[...], _DN, **_MM)
    o_ref[...] = _dinv_of(d_ref) * hw


def _tc_first(E, D, W0):
    return pl.pallas_call(
        _tc_first_body,
        grid=(NBLK,),
        in_specs=[_SPEC2, _SPECD, _SPECW],
        out_specs=_SPECR,
        out_shape=jax.ShapeDtypeStruct((NPAD, HID), jnp.float32),
    )(E, D, W0)


def _tc_step_body(p_ref, g_ref, d_ref, b_ref, w_ref, o_ref):
    dinv = _dinv_of(d_ref)
    h = jnp.maximum(dinv * (p_ref[0] + p_ref[1] + g_ref[...]) + b_ref[...], 0.0)
    o_ref[...] = dinv * lax.dot_general(h, w_ref[...], _DN, **_MM)


def _tc_step(P, g, D, b2d, Wn):
    return pl.pallas_call(
        _tc_step_body,
        grid=(NBLK,),
        in_specs=[_SPEC2, _SPECR, _SPECD, _SPECB, _SPECW],
        out_specs=_SPECR,
        out_shape=jax.ShapeDtypeStruct((NPAD, HID), jnp.float32),
    )(P, g, D, b2d, Wn)


def _tc_pool_body(p_ref, g_ref, d_ref, b_ref, bat_ref, wo_ref, bo_ref,
                  s_ref, c_ref, o_ref):
    i = pl.program_id(0)

    @pl.when(i == 0)
    def _():
        s_ref[...] = jnp.zeros_like(s_ref)
        c_ref[...] = jnp.zeros_like(c_ref)

    dinv = _dinv_of(d_ref)
    h = jnp.maximum(dinv * (p_ref[0] + p_ref[1] + g_ref[...]) + b_ref[...], 0.0)
    seg = lax.broadcasted_iota(jnp.int32, (RBLK, NG), 1)
    sel_t = (bat_ref[...] == seg).astype(jnp.float32)
    dn0 = (((0,), (0,)), ((), ()))
    s_ref[...] += lax.dot_general(sel_t, h, dn0, **_MM)
    c_ref[...] += lax.dot_general(sel_t, jnp.ones((RBLK, 1), jnp.float32),
                                  dn0, **_MM)

    @pl.when(i == NBLK - 1)
    def _():
        rep = s_ref[...] / jnp.maximum(c_ref[...], 1.0)
        o_ref[...] = lax.dot_general(rep, wo_ref[...], _DN, **_MM) + bo_ref[...]


def _tc_pool(P, g, D, b2d, bat2d, Wo, bo2d):
    _, _, out = pl.pallas_call(
        _tc_pool_body,
        grid=(NBLK,),
        in_specs=[_SPEC2, _SPECR, _SPECD, _SPECB,
                  pl.BlockSpec((RBLK, 1), lambda i: (i, 0)),
                  _SPECW, _SPECB],
        out_specs=[pl.BlockSpec((NG, HID), lambda i: (0, 0)),
                   pl.BlockSpec((NG, 1), lambda i: (0, 0)),
                   pl.BlockSpec((NG, HID), lambda i: (0, 0))],
        out_shape=[jax.ShapeDtypeStruct((NG, HID), jnp.float32),
                   jax.ShapeDtypeStruct((NG, 1), jnp.float32),
                   jax.ShapeDtypeStruct((NG, HID), jnp.float32)],
    )(P, g, D, b2d, bat2d, Wo, bo2d)
    return out


def _pack_chunks(src_idx, dst_idx, total, nchunks, fill_src_lo, fill_src_n):
    pad = NW * nchunks * CH - total
    cyc = jnp.arange(pad, dtype=jnp.int32)
    s = jnp.concatenate([src_idx, fill_src_lo + cyc % fill_src_n])
    d = jnp.concatenate([dst_idx, DUMMY_NODE + cyc % (NPAD - N_NODES)])
    return jnp.stack([s.reshape(NW, nchunks, CH),
                      d.reshape(NW, nchunks, CH)], axis=2)


def kernel(x, edge_index, batch, embs, W, b, W_out, b_out):
    x = x.astype(jnp.int32)
    src = edge_index[0].astype(jnp.int32)
    dst = edge_index[1].astype(jnp.int32)

    fidx = (x + (jnp.arange(N_FEATS, dtype=jnp.int32) * VOCAB)[None, :]).reshape(-1)
    node = jnp.repeat(jnp.arange(N_NODES, dtype=jnp.int32), N_FEATS)
    pairs = _pack_chunks(fidx, node, N_NODES * N_FEATS, K_PAIR,
                         DUMMY_TBL, TBL_ROWS - DUMMY_TBL)
    edges = _pack_chunks(src, dst, N_EDGES, K_EDGE,
                         DUMMY_NODE, NPAD - N_NODES)

    tbl = jnp.pad(embs.reshape(N_FEATS * VOCAB, HID),
                  ((0, TBL_ROWS - N_FEATS * VOCAB), (0, 0)))

    Dd = _deg_kernel()(edges)
    E = _embed_kernel()(tbl, pairs)
    g = _tc_first(E, Dd, W[0])
    batp = jnp.concatenate(
        [batch.astype(jnp.int32), jnp.full((NPAD - N_NODES,), NG, jnp.int32)]
    ).reshape(NPAD, 1)
    for l in range(3):
        P = _agg_kernel()(g, edges)
        if l < 2:
            g = _tc_step(P, g, Dd, b[l].reshape(1, HID), W[l + 1])
        else:
            out = _tc_pool(P, g, Dd, b[2].reshape(1, HID), batp,
                           W_out, b_out.reshape(1, HID))
    return out


# speedup vs baseline: 25.1658x; 1.0029x over previous
import functools

import jax
import jax.numpy as jnp
from jax import lax
from jax.experimental import pallas as pl
from jax.experimental.pallas import tpu as pltpu
from jax.experimental.pallas import tpu_sc as plsc

N_NODES = 10000
N_EDGES = 320000
N_FEATS = 9
VOCAB = 100
HID = 128
NG = 64

NC, NS = 2, 16
NW = NC * NS
CH = 128
NPAD = 10112
ROWS_PER_TILE = NPAD // NS
K_EDGE = 80
K_PAIR = 24
DUMMY_NODE = N_NODES
TBL_ROWS = 1024
DUMMY_TBL = 900


@functools.cache
def _mesh():
    return plsc.VectorSubcoreMesh(
        core_axis_name="c", subcore_axis_name="s", num_cores=NC, num_subcores=NS)


def _zero_vmem(ref, nrows, ncols, val=0.0):
    v = jnp.full((16,), val, jnp.float32)

    @pl.loop(0, nrows)
    def _(r):
        for c in range(ncols // 16):
            ref[r, pl.ds(c * 16, 16)] = v


def _fill_rows(zbuf, dst, base, nrows):
    zr = zbuf.shape[0]
    full, rem = divmod(nrows, zr)
    for k in range(full):
        pltpu.sync_copy(zbuf, dst.at[pl.ds(base + k * zr, zr)])
    if rem:
        pltpu.sync_copy(zbuf.at[pl.ds(0, rem)], dst.at[pl.ds(base + full * zr, rem)])


def _stream_gather_scatter(data_hbm, acc, idx_slab, nchunks,
                           ring, gb0, gb1, semi0, semi1, semg0, semg1):
    pltpu.async_copy(idx_slab.at[0], ring.at[0], semi0)
    pltpu.async_copy(idx_slab.at[1], ring.at[1], semi1)
    pltpu.make_async_copy(idx_slab.at[0], ring.at[0], semi0).wait()
    pltpu.async_copy(data_hbm.at[ring.at[0, 0]], gb0, semg0)

    @pl.loop(0, nchunks, step=2)
    def _(j):
        for b in (0, 1):
            jj = j + b
            gb, semg = (gb0, semg0) if b == 0 else (gb1, semg1)
            ogb, osemg = (gb1, semg1) if b == 0 else (gb0, semg0)
            semi_s = semi0 if b == 0 else semi1
            semi_o = semi1 if b == 0 else semi0

            @pl.when(jj + 1 < nchunks)
            def _():
                pltpu.make_async_copy(
                    idx_slab.at[jj + 1], ring.at[1 - b], semi_o).wait()
                pltpu.async_copy(data_hbm.at[ring.at[1 - b, 0]], ogb, osemg)

            pltpu.make_async_copy(data_hbm.at[ring.at[b, 0]], gb, semg).wait()
            pltpu.sync_copy(gb, acc.at[ring.at[b, 1]], add=True)

            @pl.when(jj + 2 < nchunks)
            def _():
                pltpu.async_copy(idx_slab.at[jj + 2], ring.at[b], semi_s)


def _deg_body(edge_hbm, d_out, ring, onz, dacc, semj0, semj1):
    ci = lax.axis_index("c")
    si = lax.axis_index("s")
    wid = ci * NS + si
    base = si * ROWS_PER_TILE
    _zero_vmem(onz.at[0], CH, HID, val=1.0)
    _zero_vmem(onz.at[1], CH, HID)
    _fill_rows(onz.at[1], dacc, base, ROWS_PER_TILE)
    plsc.subcore_barrier()
    idx_slab = edge_hbm.at[wid]
    pltpu.async_copy(idx_slab.at[0], ring.at[0], semj0)
    pltpu.async_copy(idx_slab.at[1], ring.at[1], semj1)

    @pl.loop(0, K_EDGE, step=2)
    def _(j):
        for b in (0, 1):
            jj = j + b
            sem = semj0 if b == 0 else semj1
            pltpu.make_async_copy(idx_slab.at[jj], ring.at[b], sem).wait()
            pltpu.sync_copy(onz.at[0], dacc.at[ring.at[b, 1]], add=True)

            @pl.when(jj + 2 < K_EDGE)
            def _():
                pltpu.async_copy(idx_slab.at[jj + 2], ring.at[b], sem)

    plsc.subcore_barrier()
    pltpu.sync_copy(dacc.at[pl.ds(base, ROWS_PER_TILE)],
                    d_out.at[ci, pl.ds(base, ROWS_PER_TILE)])


@functools.cache
def _deg_kernel():
    return pl.kernel(
        _deg_body,
        out_type=jax.ShapeDtypeStruct((NC, NPAD, HID), jnp.float32),
        mesh=_mesh(),
        scratch_types=[
            pltpu.VMEM((2, 2, CH), jnp.int32),
            pltpu.VMEM((2, CH, HID), jnp.float32),
            pltpu.VMEM_SHARED((NPAD, HID), jnp.float32),
            pltpu.SemaphoreType.DMA,
            pltpu.SemaphoreType.DMA,
        ],
    )


def _embed_body(tbl_hbm, pair_hbm, e_out, ring, gb0, gb1, acc,
                semi0, semi1, semg0, semg1):
    ci = lax.axis_index("c")
    si = lax.axis_index("s")
    wid = ci * NS + si
    base = si * ROWS_PER_TILE
    _zero_vmem(gb0, CH, HID)
    _fill_rows(gb0, acc, base, ROWS_PER_TILE)
    plsc.subcore_barrier()
    _stream_gather_scatter(tbl_hbm, acc, pair_hbm.at[wid], K_PAIR,
                           ring, gb0, gb1, semi0, semi1, semg0, semg1)
    plsc.subcore_barrier()
    pltpu.sync_copy(acc.at[pl.ds(base, ROWS_PER_TILE)],
                    e_out.at[ci, pl.ds(base, ROWS_PER_TILE)])


@functools.cache
def _embed_kernel():
    return pl.kernel(
        _embed_body,
        out_type=jax.ShapeDtypeStruct((NC, NPAD, HID), jnp.float32),
        mesh=_mesh(),
        scratch_types=[
            pltpu.VMEM((2, 2, CH), jnp.int32),
            pltpu.VMEM((CH, HID), jnp.float32),
            pltpu.VMEM((CH, HID), jnp.float32),
            pltpu.VMEM_SHARED((NPAD, HID), jnp.float32),
            pltpu.SemaphoreType.DMA,
            pltpu.SemaphoreType.DMA,
            pltpu.SemaphoreType.DMA,
            pltpu.SemaphoreType.DMA,
        ],
    )


def _agg_body(g_hbm, edge_hbm, p_out,
              ring, gb0, gb1, acc, semi0, semi1, semg0, semg1):
    ci = lax.axis_index("c")
    si = lax.axis_index("s")
    wid = ci * NS + si
    base = si * ROWS_PER_TILE
    _zero_vmem(gb0, CH, HID)
    _fill_rows(gb0, acc, base, ROWS_PER_TILE)
    plsc.subcore_barrier()
    _stream_gather_scatter(g_hbm, acc, edge_hbm.at[wid], K_EDGE,
                           ring, gb0, gb1, semi0, semi1, semg0, semg1)
    plsc.subcore_barrier()
    pltpu.sync_copy(acc.at[pl.ds(base, ROWS_PER_TILE)],
                    p_out.at[ci, pl.ds(base, ROWS_PER_TILE)])


@functools.cache
def _agg_kernel():
    return pl.kernel(
        _agg_body,
        out_type=jax.ShapeDtypeStruct((NC, NPAD, HID), jnp.float32),
        mesh=_mesh(),
        scratch_types=[
            pltpu.VMEM((2, 2, CH), jnp.int32),
            pltpu.VMEM((CH, HID), jnp.float32),
            pltpu.VMEM((CH, HID), jnp.float32),
            pltpu.VMEM_SHARED((NPAD, HID), jnp.float32),
            pltpu.SemaphoreType.DMA,
            pltpu.SemaphoreType.DMA,
            pltpu.SemaphoreType.DMA,
            pltpu.SemaphoreType.DMA,
        ],
    )


_MM = dict(precision=lax.Precision.HIGHEST, preferred_element_type=jnp.float32)
_DN = (((1,), (0,)), ((), ()))
NBLK = 4
RBLK = NPAD // NBLK

_SPEC2 = pl.BlockSpec((2, RBLK, HID), lambda i: (0, i, 0))
_SPECD = pl.BlockSpec((2, RBLK, HID), lambda i: (0, i, 0))
_SPECR = pl.BlockSpec((RBLK, HID), lambda i: (i, 0))
_SPECW = pl.BlockSpec((HID, HID), lambda i: (0, 0))
_SPECB = pl.BlockSpec((1, HID), lambda i: (0, 0))


def _dinv_of(d_ref):
    deg = d_ref[0][:, 0:1] + d_ref[1][:, 0:1] + 1.0
    return lax.rsqrt(deg)


def _tc_first_body(e_ref, d_ref, w_ref, o_ref):
    h = e_ref[0] + e_ref[1]
    hw = lax.dot_general(h, w_ref[...], _DN, **_MM)
    o_ref[...] = _dinv_of(d_ref) * hw


def _tc_first(E, D, W0):
    return pl.pallas_call(
        _tc_first_body,
        grid=(NBLK,),
        in_specs=[_SPEC2, _SPECD, _SPECW],
        out_specs=_SPECR,
        out_shape=jax.ShapeDtypeStruct((NPAD, HID), jnp.float32),
    )(E, D, W0)


def _tc_step_body(p_ref, g_ref, d_ref, b_ref, w_ref, o_ref):
    dinv = _dinv_of(d_ref)
    h = jnp.maximum(dinv * (p_ref[0] + p_ref[1] + g_ref[...]) + b_ref[...], 0.0)
    o_ref[...] = dinv * lax.dot_general(h, w_ref[...], _DN, **_MM)


def _tc_step(P, g, D, b2d, Wn):
    return pl.pallas_call(
        _tc_step_body,
        grid=(NBLK,),
        in_specs=[_SPEC2, _SPECR, _SPECD, _SPECB, _SPECW],
        out_specs=_SPECR,
        out_shape=jax.ShapeDtypeStruct((NPAD, HID), jnp.float32),
    )(P, g, D, b2d, Wn)


def _tc_pool_body(p_ref, g_ref, d_ref, b_ref, bat_ref, wo_ref, bo_ref,
                  s_ref, c_ref, o_ref):
    i = pl.program_id(0)

    @pl.when(i == 0)
    def _():
        s_ref[...] = jnp.zeros_like(s_ref)
        c_ref[...] = jnp.zeros_like(c_ref)

    dinv = _dinv_of(d_ref)
    h = jnp.maximum(dinv * (p_ref[0] + p_ref[1] + g_ref[...]) + b_ref[...], 0.0)
    seg = lax.broadcasted_iota(jnp.int32, (RBLK, NG), 1)
    sel_t = (bat_ref[...] == seg).astype(jnp.float32)
    dn0 = (((0,), (0,)), ((), ()))
    s_ref[...] += lax.dot_general(sel_t, h, dn0, **_MM)
    c_ref[...] += lax.dot_general(sel_t, jnp.ones((RBLK, 1), jnp.float32),
                                  dn0, **_MM)

    @pl.when(i == NBLK - 1)
    def _():
        rep = s_ref[...] / jnp.maximum(c_ref[...], 1.0)
        o_ref[...] = lax.dot_general(rep, wo_ref[...], _DN, **_MM) + bo_ref[...]


def _tc_pool(P, g, D, b2d, bat2d, Wo, bo2d):
    _, _, out = pl.pallas_call(
        _tc_pool_body,
        grid=(NBLK,),
        in_specs=[_SPEC2, _SPECR, _SPECD, _SPECB,
                  pl.BlockSpec((RBLK, 1), lambda i: (i, 0)),
                  _SPECW, _SPECB],
        out_specs=[pl.BlockSpec((NG, HID), lambda i: (0, 0)),
                   pl.BlockSpec((NG, 1), lambda i: (0, 0)),
                   pl.BlockSpec((NG, HID), lambda i: (0, 0))],
        out_shape=[jax.ShapeDtypeStruct((NG, HID), jnp.float32),
                   jax.ShapeDtypeStruct((NG, 1), jnp.float32),
                   jax.ShapeDtypeStruct((NG, HID), jnp.float32)],
    )(P, g, D, b2d, bat2d, Wo, bo2d)
    return out


def _pack_chunks(src_idx, dst_idx, total, nchunks, fill_src_lo, fill_src_n):
    pad = NW * nchunks * CH - total
    cyc = jnp.arange(pad, dtype=jnp.int32)
    s = jnp.concatenate([src_idx, fill_src_lo + cyc % fill_src_n])
    d = jnp.concatenate([dst_idx, DUMMY_NODE + cyc % (NPAD - N_NODES)])
    return jnp.stack([s.reshape(NW, nchunks, CH),
                      d.reshape(NW, nchunks, CH)], axis=2)


def kernel(x, edge_index, batch, embs, W, b, W_out, b_out):
    x = x.astype(jnp.int32)
    src = edge_index[0].astype(jnp.int32)
    dst = edge_index[1].astype(jnp.int32)

    fidx = (x + (jnp.arange(N_FEATS, dtype=jnp.int32) * VOCAB)[None, :]).reshape(-1)
    node = jnp.repeat(jnp.arange(N_NODES, dtype=jnp.int32), N_FEATS)
    pairs = _pack_chunks(fidx, node, N_NODES * N_FEATS, K_PAIR,
                         DUMMY_TBL, TBL_ROWS - DUMMY_TBL)
    edges = _pack_chunks(src, dst, N_EDGES, K_EDGE,
                         DUMMY_NODE, NPAD - N_NODES)

    tbl = jnp.pad(embs.reshape(N_FEATS * VOCAB, HID),
                  ((0, TBL_ROWS - N_FEATS * VOCAB), (0, 0)))

    Dd = _deg_kernel()(edges)
    E = _embed_kernel()(tbl, pairs)
    g = _tc_first(E, Dd, W[0])
    batp = jnp.concatenate(
        [batch.astype(jnp.int32), jnp.full((NPAD - N_NODES,), NG, jnp.int32)]
    ).reshape(NPAD, 1)
    for l in range(3):
        P = _agg_kernel()(g, edges)
        if l < 2:
            g = _tc_step(P, g, Dd, b[l].reshape(1, HID), W[l + 1])
        else:
            out = _tc_pool(P, g, Dd, b[2].reshape(1, HID), batp,
                           W_out, b_out.reshape(1, HID))
    return out
